# Initial kernel scaffold; baseline (speedup 1.0000x reference)
import jax
import jax.numpy as jnp
from jax.experimental import pallas as pl


def kernel(rgbs, W, b):
    raise NotImplementedError("write your pallas kernel here")


# speedup vs baseline: 16013.3937x; 16013.3937x over previous
import functools

import jax
import jax.numpy as jnp
from jax.experimental import pallas as pl
from jax.experimental.pallas import tpu as pltpu

_WIN = 48
_LANE = 128
_RB = 128


def _pos_body(scal_ref, x_ref, hl_ref, hr_ref, out_ref, *, K):
    RB = _RB
    R1 = RB + 1
    f32 = jnp.float32

    hl = hl_ref[...]
    hr = hr_ref[...]

    ch = [jnp.concatenate([hl[:, c, :], x_ref[c], hr[:, c, :]], axis=0)
          for c in range(3)]

    X2 = [jnp.concatenate([c_[:R1, :], c_[1:R1 + 1, :]], axis=1) for c_ in ch]

    ii = jax.lax.broadcasted_iota(jnp.int32, (256, 128), 0)
    ll = jax.lax.broadcasted_iota(jnp.int32, (256, 128), 1)
    T1 = ((ii >= ll) & (ii <= ll + (_WIN - 1))).astype(f32)
    T2 = ((ii >= ll + (_LANE - _WIN + 1)) & (ii <= ll + _LANE)).astype(f32)

    def win_sum(a):
        return jnp.dot(a, T1, preferred_element_type=f32)

    s0 = win_sum(X2[0])
    s1 = win_sum(X2[1])
    s2 = win_sum(X2[2])
    Q00 = win_sum(X2[0] * X2[0])
    Q11 = win_sum(X2[1] * X2[1])
    Q22 = win_sum(X2[2] * X2[2])
    Q01 = win_sum(X2[0] * X2[1])
    Q02 = win_sum(X2[0] * X2[2])
    Q12 = win_sum(X2[1] * X2[2])

    wn = f32(_WIN)
    a0 = wn / s0
    a1 = wn / s1
    a2 = wn / s2
    M00 = a0 * a0 * Q00 - wn
    M11 = a1 * a1 * Q11 - wn
    M22 = a2 * a2 * Q22 - wn
    M01 = a0 * a1 * Q01 - wn
    M02 = a0 * a2 * Q02 - wn
    M12 = a1 * a2 * Q12 - wn

    w00 = scal_ref[0]
    w01 = scal_ref[1]
    w02 = scal_ref[2]
    w10 = scal_ref[3]
    w11 = scal_ref[4]
    w12 = scal_ref[5]

    A2 = (w00 * w00 * M00 + w01 * w01 * M11 + w02 * w02 * M22
          + 2.0 * (w00 * w01 * M01 + w00 * w02 * M02 + w01 * w02 * M12))
    B2 = (w10 * w10 * M00 + w11 * w11 * M11 + w12 * w12 * M22
          + 2.0 * (w10 * w11 * M01 + w10 * w12 * M02 + w11 * w12 * M12))
    r = jnp.sqrt(jnp.maximum(A2, 0.0) / B2)

    g0 = w00 + r * w10
    g1 = w01 + r * w11
    g2 = w02 + r * w12
    p0 = g0 * a0
    p1 = g1 * a1
    p2 = g2 * a2
    p3 = g0 + g1 + g2

    pid = pl.program_id(0)
    rr = jax.lax.broadcasted_iota(jnp.int32, (R1, 128), 0)
    cc = jax.lax.broadcasted_iota(jnp.int32, (R1, 128), 1)
    kg = (pid * RB - 1 + rr) * _LANE + cc
    valid = (kg >= 0) & (kg < K)
    p0 = jnp.where(valid, p0, 0.0)
    p1 = jnp.where(valid, p1, 0.0)
    p2 = jnp.where(valid, p2, 0.0)
    p3 = jnp.where(valid, p3, 0.0)

    def back_sum(p):
        pr = jnp.concatenate([p[:RB, :], p[1:R1, :]], axis=1)
        return jnp.dot(pr, T2, preferred_element_type=f32)

    P0 = back_sum(p0)
    P1 = back_sum(p1)
    P2 = back_sum(p2)
    P3 = back_sum(p3)

    out_ref[...] = (ch[0][1:RB + 1, :] * P0 + ch[1][1:RB + 1, :] * P1
                    + ch[2][1:RB + 1, :] * P2 - P3)


def kernel(rgbs, W, b):
    del b
    x = rgbs[0]
    N = x.shape[0]
    K = N - _WIN
    nrows = -(-N // _LANE)
    G = -(-nrows // _RB)
    NRP = G * _RB

    xt = jnp.transpose(x).astype(jnp.float32)
    xp = jnp.pad(xt, ((0, 0), (0, NRP * _LANE - N)))
    x3 = xp.reshape(3, NRP, _LANE)
    zrow = jnp.zeros((3, 1, _LANE), jnp.float32)
    hl = jnp.concatenate([zrow, x3[:, _RB - 1::_RB, :][:, :G - 1, :]], axis=1)
    hr = jnp.concatenate([x3[:, _RB::_RB, :], zrow], axis=1)
    hl = jnp.swapaxes(hl, 0, 1)
    hr = jnp.swapaxes(hr, 0, 1)
    scal = jnp.concatenate([W[0], W[1]]).astype(jnp.float32)

    out = pl.pallas_call(
        functools.partial(_pos_body, K=K),
        grid=(G,),
        in_specs=[
            pl.BlockSpec(memory_space=pltpu.SMEM),
            pl.BlockSpec((3, _RB, _LANE), lambda g: (0, g, 0)),
            pl.BlockSpec((1, 3, _LANE), lambda g: (g, 0, 0)),
            pl.BlockSpec((1, 3, _LANE), lambda g: (g, 0, 0)),
        ],
        out_specs=pl.BlockSpec((_RB, _LANE), lambda g: (g, 0)),
        out_shape=jax.ShapeDtypeStruct((NRP, _LANE), jnp.float32),
        compiler_params=pltpu.CompilerParams(
            dimension_semantics=("parallel",)),
    )(scal, x3, hl, hr)
    return out.reshape(-1)[:N][None, :]
